# baseline (speedup 1.0000x reference)
import functools

import jax
import jax.numpy as jnp
from jax import lax
from jax.experimental import pallas as pl
from jax.experimental.pallas import tpu as pltpu
from jax.experimental.pallas import tpu_sc as plsc

DIM = 384
P = 3
CORR_DIM = 2 * 49 * P * P
E = 32768
NUM_FRAMES = 64
NUM_PATCHES = 4096

BE = 512
GRID = E // BE

NC = 2
NS = 16
NW = NC * NS
HCOL = DIM // NC

_f32 = jnp.float32
_i32 = jnp.int32


def _ln(x, g, b, eps=1e-3):
    m = jnp.mean(x, axis=-1, keepdims=True)
    v = jnp.mean((x - m) ** 2, axis=-1, keepdims=True)
    return (x - m) * jax.lax.rsqrt(v + eps) * g + b


_bf16 = jnp.bfloat16


def _mm(x, wt):
    return jax.lax.dot_general(x, wt, (((1,), (0,)), ((), ())),
                               preferred_element_type=_f32)


def _mmb(x, wt_b):
    return jax.lax.dot_general(x.astype(_bf16), wt_b, (((1,), (0,)), ((), ())),
                               preferred_element_type=_f32)


def _rowspec(d):
    return pl.BlockSpec((BE, d), lambda i: (i, 0))


def _wspec(shape):
    n = len(shape)
    return pl.BlockSpec(shape, lambda i: (0,) * n)


def _sc_mesh():
    return plsc.VectorSubcoreMesh(core_axis_name="c", subcore_axis_name="s",
                                  num_cores=NC, num_subcores=NS)


GCH = 128


def _sc_gather(table, idx):
    n_per_w = E // NW
    nch = n_per_w // GCH

    @functools.partial(
        pl.kernel,
        out_type=jax.ShapeDtypeStruct((E, DIM), _f32),
        mesh=_sc_mesh(),
        scratch_types=[
            pltpu.VMEM((n_per_w,), _i32),
            pltpu.VMEM((GCH, DIM), _f32),
            pltpu.VMEM((GCH, DIM), _f32),
            pltpu.SemaphoreType.DMA,
            pltpu.SemaphoreType.DMA,
        ],
    )
    def k(table_hbm, idx_hbm, out_hbm, idx_v, buf0, buf1, sem0, sem1):
        wid = lax.axis_index("s") * NC + lax.axis_index("c")
        base = wid * n_per_w
        pltpu.sync_copy(idx_hbm.at[pl.ds(base, n_per_w)], idx_v)
        bufs = (buf0, buf1)
        sems = (sem0, sem1)
        cps = [None, None]
        cps[0] = pltpu.async_copy(
            table_hbm.at[idx_v.at[pl.ds(0, GCH)]], buf0, sem0)
        for j in range(nch):
            nxt = (j + 1) % 2
            if j + 1 < nch:
                cps[nxt] = pltpu.async_copy(
                    table_hbm.at[idx_v.at[pl.ds((j + 1) * GCH, GCH)]],
                    bufs[nxt], sems[nxt])
            cps[j % 2].wait()
            pltpu.sync_copy(bufs[j % 2],
                            out_hbm.at[pl.ds(base + j * GCH, GCH)])

    return k(table, idx)


RPW = DIM // NS
SCH = 512


def _sc_segment_sums(euvt, seg):
    nch = E // SCH
    ngr = SCH // 16

    @functools.partial(
        pl.kernel,
        out_type=jax.ShapeDtypeStruct((2, DIM, NUM_PATCHES), _f32),
        mesh=_sc_mesh(),
        scratch_types=[
            pltpu.VMEM((RPW, NUM_PATCHES), _f32),
            pltpu.VMEM((SCH,), _i32),
            pltpu.VMEM((RPW, SCH), _f32),
        ],
        compiler_params=pltpu.CompilerParams(needs_layout_passes=False),
    )
    def k(euvt_hbm, seg_hbm, stt_hbm, tab, idx_v, buf):
        cid = lax.axis_index("c")
        sid = lax.axis_index("s")
        r0 = sid * RPW
        zero16 = jnp.zeros((16,), _f32)

        def zbody(i, _):
            for r in range(RPW):
                tab[r, pl.ds(i * 16, 16)] = zero16
            return 0
        lax.fori_loop(0, NUM_PATCHES // 16, zbody, 0)

        src = euvt_hbm.at[cid, pl.ds(r0, RPW)]

        def chunk(j, _):
            off = j * SCH
            pltpu.sync_copy(seg_hbm.at[pl.ds(off, SCH)], idx_v)
            pltpu.sync_copy(src.at[:, pl.ds(off, SCH)], buf)

            def group(g, _):
                idx16 = idx_v[pl.ds(g * 16, 16)]
                for c in range(RPW):
                    vals = buf[c, pl.ds(g * 16, 16)]
                    plsc.addupdate_scatter(
                        tab, [jnp.full((16,), c, _i32), idx16], vals)
                return 0
            lax.fori_loop(0, ngr, group, 0)
            return 0
        lax.fori_loop(0, nch, chunk, 0)
        pltpu.sync_copy(tab, stt_hbm.at[cid, pl.ds(r0, RPW)])

    return k(euvt, seg)


def _enc_body(corr_ref, net_ref, inp_ref, w0, b0, w1, b1, lng, lnb, w2, b2,
              ng, nb, out_ref):
    c = jnp.maximum(_mmb(corr_ref[...], w0[...]) + b0[...], 0.0)
    c = _mmb(c, w1[...]) + b1[...]
    c = jnp.maximum(_ln(c, lng[...], lnb[...]), 0.0)
    c = _mmb(c, w2[...]) + b2[...]
    x = net_ref[...] + inp_ref[...] + c
    out_ref[...] = _ln(x, ng[...], nb[...])


def _encoder(corr, net, inp, p):
    args = [corr, net, inp,
            p['corr_0w'].T.astype(_bf16), p['corr_0b'][None],
            p['corr_1w'].T.astype(_bf16), p['corr_1b'][None],
            p['corr_ln_g'][None], p['corr_ln_b'][None],
            p['corr_2w'].T.astype(_bf16), p['corr_2b'][None],
            p['norm_g'][None], p['norm_b'][None]]
    specs = [_rowspec(CORR_DIM), _rowspec(DIM), _rowspec(DIM)] + \
            [_wspec(a.shape) for a in args[3:]]
    return pl.pallas_call(
        _enc_body,
        grid=(GRID,),
        in_specs=specs,
        out_specs=_rowspec(DIM),
        out_shape=jax.ShapeDtypeStruct((E, DIM), _f32),
    )(*args)


def _mlp_res_body(base_ref, xg_ref, m_ref, w0, b0, w1, b1, out_ref):
    xg = xg_ref[...] * (m_ref[...] >= 0).astype(_f32)
    h = jnp.maximum(_mmb(xg, w0[...]) + b0[...], 0.0)
    out_ref[...] = base_ref[...] + _mmb(h, w1[...]) + b1[...]


def _mlp_residual(base, xg, msk, p, name):
    args = [base, xg, msk, p[name + '_0w'].T.astype(_bf16), p[name + '_0b'][None],
            p[name + '_1w'].T.astype(_bf16), p[name + '_1b'][None]]
    specs = [_rowspec(DIM), _rowspec(DIM), _rowspec(1)] + \
            [_wspec(a.shape) for a in args[3:]]
    return pl.pallas_call(
        _mlp_res_body,
        grid=(GRID,),
        in_specs=specs,
        out_specs=_rowspec(DIM),
        out_shape=jax.ShapeDtypeStruct((E, DIM), _f32),
    )(*args)


def _euf_bodies(nadd):
    def body(*refs):
        xs = refs[:1 + nadd]
        gw, gb, fw, fb, eu_ref = refs[1 + nadd:]
        x = xs[0][...]
        for r in xs[1:]:
            x = x + r[...]
        xt = x.astype(_bf16).T
        et = jnp.exp(jax.lax.dot_general(
            gw[...], xt, (((1,), (0,)), ((), ())),
            preferred_element_type=_f32) + gb[...])
        ft = jax.lax.dot_general(
            fw[...], xt, (((1,), (0,)), ((), ())),
            preferred_element_type=_f32) + fb[...]
        eu_ref[0] = et
        eu_ref[1] = ft * et
    return body


def _euf(xs, pa):
    args = list(xs) + [pa['g_w'].astype(_bf16), pa['g_b'][:, None],
                       pa['f_w'].astype(_bf16), pa['f_b'][:, None]]
    specs = [_rowspec(DIM)] * len(xs) + [_wspec(a.shape) for a in args[len(xs):]]
    return pl.pallas_call(
        _euf_bodies(len(xs) - 1),
        grid=(GRID,),
        in_specs=specs,
        out_specs=pl.BlockSpec((2, DIM, BE), lambda i: (0, 0, i)),
        out_shape=jax.ShapeDtypeStruct((2, DIM, E), _f32),
    )(*args)


BH = 512


def _h_body(stt_ref, hw, hb, out_ref):
    yt = (stt_ref[1] / jnp.maximum(stt_ref[0], 1e-30)).astype(_bf16)
    out_ref[...] = (jax.lax.dot_general(
        hw[...], yt, (((1,), (0,)), ((), ())),
        preferred_element_type=_f32) + hb[...]).T


def _hproj(stt, pa):
    args = [stt, pa['h_w'].astype(_bf16), pa['h_b'][:, None]]
    specs = [pl.BlockSpec((2, DIM, BH), lambda i: (0, 0, i))] + \
            [_wspec(a.shape) for a in args[1:]]
    return pl.pallas_call(
        _h_body,
        grid=(NUM_PATCHES // BH,),
        in_specs=specs,
        out_specs=pl.BlockSpec((BH, DIM), lambda i: (i, 0)),
        out_shape=jax.ShapeDtypeStruct((NUM_PATCHES, DIM), _f32),
    )(*args)


def _gru_body(x_ref, a_ref, b_ref, l1g, l1b, f1w, f1b, g1w, g1b, l2g, l2b,
              f2w, f2b, g2w, g2b, dw, db, ww, wb, net_ref, d_ref, w_ref):
    x = x_ref[...] + a_ref[...] + b_ref[...]
    x = _ln(x, l1g[...], l1b[...])
    f = jax.nn.sigmoid(_mm(x, f1w[...]) + f1b[...])
    g = jnp.maximum(_mm(x, g1w[...]) + g1b[...], 0.0)
    x = x * f + g
    x = _ln(x, l2g[...], l2b[...])
    f = jax.nn.sigmoid(_mm(x, f2w[...]) + f2b[...])
    g = jnp.maximum(_mm(x, g2w[...]) + g2b[...], 0.0)
    x = x * f + g
    net_ref[...] = x
    r = jnp.maximum(x, 0.0)
    d_ref[...] = _mm(r, dw[...]) + db[...]
    w_ref[...] = jax.nn.sigmoid(_mm(r, ww[...]) + wb[...])


def _gru(x, ha, hb_, p):
    args = [x, ha, hb_,
            p['gru_ln1_g'][None], p['gru_ln1_b'][None],
            p['gru_gr1']['f_w'].T, p['gru_gr1']['f_b'][None],
            p['gru_gr1']['g_w'].T, p['gru_gr1']['g_b'][None],
            p['gru_ln2_g'][None], p['gru_ln2_b'][None],
            p['gru_gr2']['f_w'].T, p['gru_gr2']['f_b'][None],
            p['gru_gr2']['g_w'].T, p['gru_gr2']['g_b'][None],
            p['d_w'].T, p['d_b'][None], p['w_w'].T, p['w_b'][None]]
    specs = [_rowspec(DIM)] * 3 + [_wspec(a.shape) for a in args[3:]]
    return pl.pallas_call(
        _gru_body,
        grid=(GRID,),
        in_specs=specs,
        out_specs=(_rowspec(DIM), _rowspec(2), _rowspec(2)),
        out_shape=(jax.ShapeDtypeStruct((E, DIM), _f32),
                   jax.ShapeDtypeStruct((E, 2), _f32),
                   jax.ShapeDtypeStruct((E, 2), _f32)),
    )(*args)


def _soft_agg(xs, seg, pa):
    euv = _euf(xs, pa)
    st = _sc_segment_sums(euv, seg)
    h = _hproj(st, pa)
    return _sc_gather(h, seg)


def kernel(net, inp, corr, ii, jj, kk, params):
    p = params
    netr = net[0]
    inpr = inp[0]
    corrr = corr[0]

    sz = NUM_PATCHES * NUM_FRAMES
    tab = jnp.full((sz,), -1, dtype=_i32)
    tab = tab.at[kk * NUM_FRAMES + jj].set(jnp.arange(E, dtype=_i32))
    kp = jnp.clip(kk * NUM_FRAMES + (jj - 1), 0, sz - 1)
    kn = jnp.clip(kk * NUM_FRAMES + (jj + 1), 0, sz - 1)
    ix = jnp.where(jj - 1 >= 0, tab[kp], -1)
    jx = jnp.where(jj + 1 < NUM_FRAMES, tab[kn], -1)

    net1 = _encoder(corrr, netr, inpr, p)

    net2 = _mlp_residual(net1, _sc_gather(net1, jnp.maximum(ix, 0)),
                         ix[:, None], p, 'c1')
    net3 = _mlp_residual(net2, _sc_gather(net2, jnp.maximum(jx, 0)),
                         jx[:, None], p, 'c2')

    ha = _soft_agg([net3], kk, p['agg_kk'])
    seg_ij = ii * NUM_FRAMES + jj
    hb_ = _soft_agg([net3, ha], seg_ij, p['agg_ij'])

    net_out, d, w = _gru(net3, ha, hb_, p)
    return net_out[None], d[None], w[None]


# speedup vs baseline: 1.0022x; 1.0022x over previous
import functools

import jax
import jax.numpy as jnp
from jax import lax
from jax.experimental import pallas as pl
from jax.experimental.pallas import tpu as pltpu
from jax.experimental.pallas import tpu_sc as plsc

DIM = 384
P = 3
CORR_DIM = 2 * 49 * P * P
E = 32768
NUM_FRAMES = 64
NUM_PATCHES = 4096

BE = 512
GRID = E // BE

NC = 2
NS = 16
NW = NC * NS
HCOL = DIM // NC

_f32 = jnp.float32
_i32 = jnp.int32


def _ln(x, g, b, eps=1e-3):
    m = jnp.mean(x, axis=-1, keepdims=True)
    v = jnp.mean((x - m) ** 2, axis=-1, keepdims=True)
    return (x - m) * jax.lax.rsqrt(v + eps) * g + b


_bf16 = jnp.bfloat16


def _mm(x, wt):
    return jax.lax.dot_general(x, wt, (((1,), (0,)), ((), ())),
                               preferred_element_type=_f32)


def _mmb(x, wt_b):
    return jax.lax.dot_general(x.astype(_bf16), wt_b, (((1,), (0,)), ((), ())),
                               preferred_element_type=_f32)


def _rowspec(d):
    return pl.BlockSpec((BE, d), lambda i: (i, 0))


def _wspec(shape):
    n = len(shape)
    return pl.BlockSpec(shape, lambda i: (0,) * n)


def _sc_mesh():
    return plsc.VectorSubcoreMesh(core_axis_name="c", subcore_axis_name="s",
                                  num_cores=NC, num_subcores=NS)


GSUB = 32
GK = 8


def _sc_gather(table, idx):
    n_per_w = E // NW
    wave = GSUB * GK
    nwv = n_per_w // wave

    @functools.partial(
        pl.kernel,
        out_type=jax.ShapeDtypeStruct((E, DIM), _f32),
        mesh=_sc_mesh(),
        scratch_types=[
            pltpu.VMEM((n_per_w,), _i32),
            pltpu.VMEM((wave, DIM), _f32),
            pltpu.SemaphoreType.DMA,
        ],
    )
    def k(table_hbm, idx_hbm, out_hbm, idx_v, buf, sem):
        wid = lax.axis_index("s") * NC + lax.axis_index("c")
        base = wid * n_per_w
        pltpu.sync_copy(idx_hbm.at[pl.ds(base, n_per_w)], idx_v)
        for w in range(nwv):
            o = w * wave
            cps = [
                pltpu.async_copy(
                    table_hbm.at[idx_v.at[pl.ds(o + s * GSUB, GSUB)]],
                    buf.at[pl.ds(s * GSUB, GSUB)], sem)
                for s in range(GK)
            ]
            for cp in cps:
                cp.wait()
            pltpu.sync_copy(buf, out_hbm.at[pl.ds(base + o, wave)])

    return k(table, idx)


RPW = DIM // NS
SCH = 512


def _sc_segment_sums(euvt, seg):
    nch = E // SCH
    ngr = SCH // 16

    @functools.partial(
        pl.kernel,
        out_type=jax.ShapeDtypeStruct((2, DIM, NUM_PATCHES), _f32),
        mesh=_sc_mesh(),
        scratch_types=[
            pltpu.VMEM((RPW, NUM_PATCHES), _f32),
            pltpu.VMEM((SCH,), _i32),
            pltpu.VMEM((RPW, SCH), _f32),
        ],
        compiler_params=pltpu.CompilerParams(needs_layout_passes=False),
    )
    def k(euvt_hbm, seg_hbm, stt_hbm, tab, idx_v, buf):
        cid = lax.axis_index("c")
        sid = lax.axis_index("s")
        r0 = sid * RPW
        zero16 = jnp.zeros((16,), _f32)

        def zbody(i, _):
            for r in range(RPW):
                tab[r, pl.ds(i * 16, 16)] = zero16
            return 0
        lax.fori_loop(0, NUM_PATCHES // 16, zbody, 0)

        src = euvt_hbm.at[cid, pl.ds(r0, RPW)]

        def chunk(j, _):
            off = j * SCH
            pltpu.sync_copy(seg_hbm.at[pl.ds(off, SCH)], idx_v)
            pltpu.sync_copy(src.at[:, pl.ds(off, SCH)], buf)

            def group(g, _):
                idx16 = idx_v[pl.ds(g * 16, 16)]
                for c in range(RPW):
                    vals = buf[c, pl.ds(g * 16, 16)]
                    plsc.addupdate_scatter(
                        tab, [jnp.full((16,), c, _i32), idx16], vals)
                return 0
            lax.fori_loop(0, ngr, group, 0)
            return 0
        lax.fori_loop(0, nch, chunk, 0)
        pltpu.sync_copy(tab, stt_hbm.at[cid, pl.ds(r0, RPW)])

    return k(euvt, seg)


def _enc_body(corr_ref, net_ref, inp_ref, w0, b0, w1, b1, lng, lnb, w2, b2,
              ng, nb, out_ref):
    c = jnp.maximum(_mmb(corr_ref[...], w0[...]) + b0[...], 0.0)
    c = _mmb(c, w1[...]) + b1[...]
    c = jnp.maximum(_ln(c, lng[...], lnb[...]), 0.0)
    c = _mmb(c, w2[...]) + b2[...]
    x = net_ref[...] + inp_ref[...] + c
    out_ref[...] = _ln(x, ng[...], nb[...])


def _encoder(corr, net, inp, p):
    args = [corr, net, inp,
            p['corr_0w'].T.astype(_bf16), p['corr_0b'][None],
            p['corr_1w'].T.astype(_bf16), p['corr_1b'][None],
            p['corr_ln_g'][None], p['corr_ln_b'][None],
            p['corr_2w'].T.astype(_bf16), p['corr_2b'][None],
            p['norm_g'][None], p['norm_b'][None]]
    specs = [_rowspec(CORR_DIM), _rowspec(DIM), _rowspec(DIM)] + \
            [_wspec(a.shape) for a in args[3:]]
    return pl.pallas_call(
        _enc_body,
        grid=(GRID,),
        in_specs=specs,
        out_specs=_rowspec(DIM),
        out_shape=jax.ShapeDtypeStruct((E, DIM), _f32),
    )(*args)


def _mlp_res_body(base_ref, xg_ref, m_ref, w0, b0, w1, b1, out_ref):
    xg = xg_ref[...] * (m_ref[...] >= 0).astype(_f32)
    h = jnp.maximum(_mmb(xg, w0[...]) + b0[...], 0.0)
    out_ref[...] = base_ref[...] + _mmb(h, w1[...]) + b1[...]


def _mlp_residual(base, xg, msk, p, name):
    args = [base, xg, msk, p[name + '_0w'].T.astype(_bf16), p[name + '_0b'][None],
            p[name + '_1w'].T.astype(_bf16), p[name + '_1b'][None]]
    specs = [_rowspec(DIM), _rowspec(DIM), _rowspec(1)] + \
            [_wspec(a.shape) for a in args[3:]]
    return pl.pallas_call(
        _mlp_res_body,
        grid=(GRID,),
        in_specs=specs,
        out_specs=_rowspec(DIM),
        out_shape=jax.ShapeDtypeStruct((E, DIM), _f32),
    )(*args)


def _euf_bodies(nadd):
    def body(*refs):
        xs = refs[:1 + nadd]
        gw, gb, fw, fb, eu_ref = refs[1 + nadd:]
        x = xs[0][...]
        for r in xs[1:]:
            x = x + r[...]
        xt = x.astype(_bf16).T
        et = jnp.exp(jax.lax.dot_general(
            gw[...], xt, (((1,), (0,)), ((), ())),
            preferred_element_type=_f32) + gb[...])
        ft = jax.lax.dot_general(
            fw[...], xt, (((1,), (0,)), ((), ())),
            preferred_element_type=_f32) + fb[...]
        eu_ref[0] = et
        eu_ref[1] = ft * et
    return body


def _euf(xs, pa):
    args = list(xs) + [pa['g_w'].astype(_bf16), pa['g_b'][:, None],
                       pa['f_w'].astype(_bf16), pa['f_b'][:, None]]
    specs = [_rowspec(DIM)] * len(xs) + [_wspec(a.shape) for a in args[len(xs):]]
    return pl.pallas_call(
        _euf_bodies(len(xs) - 1),
        grid=(GRID,),
        in_specs=specs,
        out_specs=pl.BlockSpec((2, DIM, BE), lambda i: (0, 0, i)),
        out_shape=jax.ShapeDtypeStruct((2, DIM, E), _f32),
    )(*args)


BH = 512


def _h_body(stt_ref, hw, hb, out_ref):
    yt = (stt_ref[1] / jnp.maximum(stt_ref[0], 1e-30)).astype(_bf16)
    out_ref[...] = (jax.lax.dot_general(
        hw[...], yt, (((1,), (0,)), ((), ())),
        preferred_element_type=_f32) + hb[...]).T


def _hproj(stt, pa):
    args = [stt, pa['h_w'].astype(_bf16), pa['h_b'][:, None]]
    specs = [pl.BlockSpec((2, DIM, BH), lambda i: (0, 0, i))] + \
            [_wspec(a.shape) for a in args[1:]]
    return pl.pallas_call(
        _h_body,
        grid=(NUM_PATCHES // BH,),
        in_specs=specs,
        out_specs=pl.BlockSpec((BH, DIM), lambda i: (i, 0)),
        out_shape=jax.ShapeDtypeStruct((NUM_PATCHES, DIM), _f32),
    )(*args)


def _gru_body(x_ref, a_ref, b_ref, l1g, l1b, f1w, f1b, g1w, g1b, l2g, l2b,
              f2w, f2b, g2w, g2b, dw, db, ww, wb, net_ref, d_ref, w_ref):
    x = x_ref[...] + a_ref[...] + b_ref[...]
    x = _ln(x, l1g[...], l1b[...])
    f = jax.nn.sigmoid(_mm(x, f1w[...]) + f1b[...])
    g = jnp.maximum(_mm(x, g1w[...]) + g1b[...], 0.0)
    x = x * f + g
    x = _ln(x, l2g[...], l2b[...])
    f = jax.nn.sigmoid(_mm(x, f2w[...]) + f2b[...])
    g = jnp.maximum(_mm(x, g2w[...]) + g2b[...], 0.0)
    x = x * f + g
    net_ref[...] = x
    r = jnp.maximum(x, 0.0)
    d_ref[...] = _mm(r, dw[...]) + db[...]
    w_ref[...] = jax.nn.sigmoid(_mm(r, ww[...]) + wb[...])


def _gru(x, ha, hb_, p):
    args = [x, ha, hb_,
            p['gru_ln1_g'][None], p['gru_ln1_b'][None],
            p['gru_gr1']['f_w'].T, p['gru_gr1']['f_b'][None],
            p['gru_gr1']['g_w'].T, p['gru_gr1']['g_b'][None],
            p['gru_ln2_g'][None], p['gru_ln2_b'][None],
            p['gru_gr2']['f_w'].T, p['gru_gr2']['f_b'][None],
            p['gru_gr2']['g_w'].T, p['gru_gr2']['g_b'][None],
            p['d_w'].T, p['d_b'][None], p['w_w'].T, p['w_b'][None]]
    specs = [_rowspec(DIM)] * 3 + [_wspec(a.shape) for a in args[3:]]
    return pl.pallas_call(
        _gru_body,
        grid=(GRID,),
        in_specs=specs,
        out_specs=(_rowspec(DIM), _rowspec(2), _rowspec(2)),
        out_shape=(jax.ShapeDtypeStruct((E, DIM), _f32),
                   jax.ShapeDtypeStruct((E, 2), _f32),
                   jax.ShapeDtypeStruct((E, 2), _f32)),
    )(*args)


def _soft_agg(xs, seg, pa):
    euv = _euf(xs, pa)
    st = _sc_segment_sums(euv, seg)
    h = _hproj(st, pa)
    return _sc_gather(h, seg)


def kernel(net, inp, corr, ii, jj, kk, params):
    p = params
    netr = net[0]
    inpr = inp[0]
    corrr = corr[0]

    sz = NUM_PATCHES * NUM_FRAMES
    tab = jnp.full((sz,), -1, dtype=_i32)
    tab = tab.at[kk * NUM_FRAMES + jj].set(jnp.arange(E, dtype=_i32))
    kp = jnp.clip(kk * NUM_FRAMES + (jj - 1), 0, sz - 1)
    kn = jnp.clip(kk * NUM_FRAMES + (jj + 1), 0, sz - 1)
    ix = jnp.where(jj - 1 >= 0, tab[kp], -1)
    jx = jnp.where(jj + 1 < NUM_FRAMES, tab[kn], -1)

    net1 = _encoder(corrr, netr, inpr, p)

    net2 = _mlp_residual(net1, _sc_gather(net1, jnp.maximum(ix, 0)),
                         ix[:, None], p, 'c1')
    net3 = _mlp_residual(net2, _sc_gather(net2, jnp.maximum(jx, 0)),
                         jx[:, None], p, 'c2')

    ha = _soft_agg([net3], kk, p['agg_kk'])
    seg_ij = ii * NUM_FRAMES + jj
    hb_ = _soft_agg([net3, ha], seg_ij, p['agg_ij'])

    net_out, d, w = _gru(net3, ha, hb_, p)
    return net_out[None], d[None], w[None]
